# Initial kernel scaffold; baseline (speedup 1.0000x reference)
import jax
import jax.numpy as jnp
from jax.experimental import pallas as pl


def kernel(x_nchw, stem_w, stem_b, exp_w, exp_b, dw_w, dw_b, se_r_w, se_r_b, se_e_w, se_e_b, proj_w, proj_b, head_w, head_b, fc_w, fc_b):
    raise NotImplementedError("write your pallas kernel here")


# speedup vs baseline: 1.8021x; 1.8021x over previous
import functools

import jax
import jax.numpy as jnp
from jax.experimental import pallas as pl
from jax.experimental.pallas import tpu as pltpu


def _fused_net_kernel(cols_ref, ws_ref, bs_ref, we_ref, be_ref,
                      wdw_ref, bdw_ref, w1_ref, b1_ref, w2_ref, b2_ref,
                      wp_ref, bp_ref, wh_ref, bh_ref, wf_ref, bf_ref,
                      o_ref, *, Ho, Wo):
    S = Ho * Wo
    inv_s = 1.0 / S

    cols = cols_ref[0]
    h = jnp.dot(cols, ws_ref[...],
                preferred_element_type=jnp.float32) + bs_ref[...]
    h = h * jax.nn.sigmoid(h)
    hb = h.astype(jnp.bfloat16)

    e = jnp.dot(hb, we_ref[...],
                preferred_element_type=jnp.float32) + be_ref[...]
    e = e * jax.nn.sigmoid(e)
    C = we_ref.shape[1]
    eb = e.astype(jnp.bfloat16).reshape(Ho, Wo, C)

    zr = jnp.zeros((1, Wo, C), jnp.bfloat16)
    ep = jnp.concatenate([zr, eb, zr], axis=0)
    zc = jnp.zeros((Ho + 2, 1, C), jnp.bfloat16)
    ep = jnp.concatenate([zc, ep, zc], axis=1)
    shifted = [ep[:, j:j + Wo, :] for j in range(3)]
    acc = jnp.zeros((Ho, Wo, C), jnp.float32)
    for i in range(3):
        for j in range(3):
            acc = acc + (shifted[j][i:i + Ho] * wdw_ref[3 * i + j]
                         ).astype(jnp.float32)
    y = acc + bdw_ref[...]
    y = y * jax.nn.sigmoid(y)

    pooled = jnp.sum(jnp.sum(y, axis=0), axis=0, keepdims=True) * inv_s
    r = jnp.dot(pooled.astype(jnp.bfloat16), w1_ref[...],
                preferred_element_type=jnp.float32) + b1_ref[...]
    r = r * jax.nn.sigmoid(r)
    g = jax.nn.sigmoid(jnp.dot(r.astype(jnp.bfloat16), w2_ref[...],
                               preferred_element_type=jnp.float32)
                       + b2_ref[...])

    dg = (y.astype(jnp.bfloat16).reshape(S, C) * g).astype(jnp.bfloat16)
    hn = (jnp.dot(dg, wp_ref[...], preferred_element_type=jnp.float32)
          + bp_ref[...] + hb.astype(jnp.float32))
    hd = (jnp.dot(hn.astype(jnp.bfloat16), wh_ref[...],
                  preferred_element_type=jnp.float32) + bh_ref[...])
    hd = hd * jax.nn.sigmoid(hd)
    p2 = jnp.sum(hd, axis=0, keepdims=True) * inv_s
    logits = (jnp.dot(p2.astype(jnp.bfloat16), wf_ref[...],
                      preferred_element_type=jnp.float32) + bf_ref[...])
    o_ref[0] = logits


def kernel(x_nchw, stem_w, stem_b, exp_w, exp_b, dw_w, dw_b,
           se_r_w, se_r_b, se_e_w, se_e_b, proj_w, proj_b,
           head_w, head_b, fc_w, fc_b):
    B, _, H, W = x_nchw.shape
    Ho, Wo = H // 2, W // 2
    S = Ho * Wo
    Cs = stem_w.shape[1]
    C = exp_w.shape[1]
    Cse = se_r_w.shape[1]
    Ch = head_w.shape[1]
    NC = fc_w.shape[1]
    K = stem_w.shape[0]

    x = jnp.transpose(x_nchw, (0, 2, 3, 1)).astype(jnp.bfloat16)
    xp = jnp.pad(x, ((0, 0), (0, 1), (0, 1), (0, 0)))
    taps = [xp[:, i:i + 2 * Ho:2, j:j + 2 * Wo:2, :]
            for i in range(3) for j in range(3)]
    cols = jnp.concatenate(taps, axis=-1).reshape(B, S, K)

    out = pl.pallas_call(
        functools.partial(_fused_net_kernel, Ho=Ho, Wo=Wo),
        out_shape=jax.ShapeDtypeStruct((B, 1, NC), jnp.float32),
        grid=(B,),
        in_specs=[
            pl.BlockSpec((1, S, K), lambda b: (b, 0, 0)),
            pl.BlockSpec((K, Cs), lambda b: (0, 0)),
            pl.BlockSpec((1, Cs), lambda b: (0, 0)),
            pl.BlockSpec((Cs, C), lambda b: (0, 0)),
            pl.BlockSpec((1, C), lambda b: (0, 0)),
            pl.BlockSpec((9, C), lambda b: (0, 0)),
            pl.BlockSpec((1, C), lambda b: (0, 0)),
            pl.BlockSpec((C, Cse), lambda b: (0, 0)),
            pl.BlockSpec((1, Cse), lambda b: (0, 0)),
            pl.BlockSpec((Cse, C), lambda b: (0, 0)),
            pl.BlockSpec((1, C), lambda b: (0, 0)),
            pl.BlockSpec((C, Cs), lambda b: (0, 0)),
            pl.BlockSpec((1, Cs), lambda b: (0, 0)),
            pl.BlockSpec((Cs, Ch), lambda b: (0, 0)),
            pl.BlockSpec((1, Ch), lambda b: (0, 0)),
            pl.BlockSpec((Ch, NC), lambda b: (0, 0)),
            pl.BlockSpec((1, NC), lambda b: (0, 0)),
        ],
        out_specs=pl.BlockSpec((1, 1, NC), lambda b: (b, 0, 0)),
        compiler_params=pltpu.CompilerParams(dimension_semantics=("parallel",)),
    )(cols, stem_w, stem_b.reshape(1, Cs).astype(jnp.float32),
      exp_w, exp_b.reshape(1, C).astype(jnp.float32),
      dw_w, dw_b.reshape(1, C).astype(jnp.float32),
      se_r_w, se_r_b.reshape(1, Cse).astype(jnp.float32),
      se_e_w, se_e_b.reshape(1, C).astype(jnp.float32),
      proj_w, proj_b.reshape(1, Cs).astype(jnp.float32),
      head_w, head_b.reshape(1, Ch).astype(jnp.float32),
      fc_w, fc_b.reshape(1, NC).astype(jnp.float32))
    return out.reshape(B, NC)


# speedup vs baseline: 2.6563x; 1.4740x over previous
import functools

import jax
import jax.numpy as jnp
from jax.experimental import pallas as pl
from jax.experimental.pallas import tpu as pltpu


def _fused_net_kernel(cols_ref, ws_ref, bs_ref, we_ref, be_ref,
                      wdw_ref, bdw_ref, w1_ref, b1_ref, w2_ref, b2_ref,
                      wp_ref, bp_ref, wh_ref, bh_ref, wf_ref, bf_ref,
                      o_ref, *, Ho, Wo):
    S = Ho * Wo
    inv_s = 1.0 / S

    cols = cols_ref[0]
    h = jnp.dot(cols, ws_ref[...],
                preferred_element_type=jnp.float32) + bs_ref[...]
    h = h * jax.nn.sigmoid(h)
    hb = h.astype(jnp.bfloat16)

    e = jnp.dot(hb, we_ref[...],
                preferred_element_type=jnp.float32) + be_ref[...]
    e = e * jax.nn.sigmoid(e)
    C2 = we_ref.shape[1]
    eb = e.astype(jnp.bfloat16).reshape(Ho, Wo, C2)

    zr = jnp.zeros((1, Wo, C2), jnp.bfloat16)
    ep = jnp.concatenate([zr, eb, zr], axis=0)
    zc = jnp.zeros((Ho + 2, 1, C2), jnp.bfloat16)
    ep = jnp.concatenate([zc, ep, zc], axis=1)
    shifted = [ep[:, j:j + Wo, :] for j in range(3)]
    acc = jnp.zeros((Ho, Wo, C2), jnp.float32)
    for i in range(3):
        for j in range(3):
            acc = acc + (shifted[j][i:i + Ho] * wdw_ref[3 * i + j]
                         ).astype(jnp.float32)
    y = acc + bdw_ref[...]
    y = y * jax.nn.sigmoid(y)

    pooled = jnp.sum(jnp.sum(y, axis=0), axis=0, keepdims=True) * inv_s
    r = jnp.dot(pooled.astype(jnp.bfloat16), w1_ref[...],
                preferred_element_type=jnp.float32) + b1_ref[...]
    r = r * jax.nn.sigmoid(r)
    g = jax.nn.sigmoid(jnp.dot(r.astype(jnp.bfloat16), w2_ref[...],
                               preferred_element_type=jnp.float32)
                       + b2_ref[...])

    dg = (y.astype(jnp.bfloat16).reshape(S, C2) * g).astype(jnp.bfloat16)
    hn = (jnp.dot(dg, wp_ref[...], preferred_element_type=jnp.float32)
          + bp_ref[...] + hb.astype(jnp.float32))
    hd = (jnp.dot(hn.astype(jnp.bfloat16), wh_ref[...],
                  preferred_element_type=jnp.float32) + bh_ref[...])
    hd = hd * jax.nn.sigmoid(hd)
    p2 = jnp.sum(hd, axis=0, keepdims=True) * inv_s
    logits = (jnp.dot(p2.astype(jnp.bfloat16), wf_ref[...],
                      preferred_element_type=jnp.float32) + bf_ref[...])
    o_ref[0] = logits


def _blockdiag2(w):
    K, N = w.shape
    z = jnp.zeros((K, N), w.dtype)
    return jnp.concatenate(
        [jnp.concatenate([w, z], axis=1), jnp.concatenate([z, w], axis=1)],
        axis=0)


def _pair2(v):
    return jnp.tile(v.reshape(1, -1).astype(jnp.float32), (1, 2))


def kernel(x_nchw, stem_w, stem_b, exp_w, exp_b, dw_w, dw_b,
           se_r_w, se_r_b, se_e_w, se_e_b, proj_w, proj_b,
           head_w, head_b, fc_w, fc_b):
    B, C_IN, H, W = x_nchw.shape
    Ho, Wo = H // 2, W // 2
    S = Ho * Wo
    P = B // 2
    Cs = stem_w.shape[1]
    C = exp_w.shape[1]
    Cse = se_r_w.shape[1]
    Ch = head_w.shape[1]
    NC = fc_w.shape[1]

    x = jnp.transpose(x_nchw, (0, 2, 3, 1)).astype(jnp.bfloat16)
    xp = jnp.pad(x, ((0, 0), (0, 1), (0, 1), (0, 0)))
    x2 = jnp.concatenate([xp[0::2], xp[1::2]], axis=-1)
    taps = [x2[:, i:i + 2 * Ho:2, j:j + 2 * Wo:2, :]
            for i in range(3) for j in range(3)]
    cols = jnp.concatenate(taps, axis=-1).reshape(P, S, 9 * 2 * C_IN)

    ws3 = stem_w.reshape(9, C_IN, Cs)
    z3 = jnp.zeros_like(ws3)
    ws2 = jnp.concatenate(
        [jnp.concatenate([ws3, z3], axis=2),
         jnp.concatenate([z3, ws3], axis=2)], axis=1).reshape(9 * 2 * C_IN,
                                                              2 * Cs)
    we2 = _blockdiag2(exp_w)
    wdw2 = jnp.tile(dw_w, (1, 2))
    w1_2 = _blockdiag2(se_r_w)
    w2_2 = _blockdiag2(se_e_w)
    wp2 = _blockdiag2(proj_w)
    wh2 = _blockdiag2(head_w)
    wf2 = _blockdiag2(fc_w)

    out = pl.pallas_call(
        functools.partial(_fused_net_kernel, Ho=Ho, Wo=Wo),
        out_shape=jax.ShapeDtypeStruct((P, 1, 2 * NC), jnp.float32),
        grid=(P,),
        in_specs=[
            pl.BlockSpec((1, S, 9 * 2 * C_IN), lambda b: (b, 0, 0)),
            pl.BlockSpec((9 * 2 * C_IN, 2 * Cs), lambda b: (0, 0)),
            pl.BlockSpec((1, 2 * Cs), lambda b: (0, 0)),
            pl.BlockSpec((2 * Cs, 2 * C), lambda b: (0, 0)),
            pl.BlockSpec((1, 2 * C), lambda b: (0, 0)),
            pl.BlockSpec((9, 2 * C), lambda b: (0, 0)),
            pl.BlockSpec((1, 2 * C), lambda b: (0, 0)),
            pl.BlockSpec((2 * C, 2 * Cse), lambda b: (0, 0)),
            pl.BlockSpec((1, 2 * Cse), lambda b: (0, 0)),
            pl.BlockSpec((2 * Cse, 2 * C), lambda b: (0, 0)),
            pl.BlockSpec((1, 2 * C), lambda b: (0, 0)),
            pl.BlockSpec((2 * C, 2 * Cs), lambda b: (0, 0)),
            pl.BlockSpec((1, 2 * Cs), lambda b: (0, 0)),
            pl.BlockSpec((2 * Cs, 2 * Ch), lambda b: (0, 0)),
            pl.BlockSpec((1, 2 * Ch), lambda b: (0, 0)),
            pl.BlockSpec((2 * Ch, 2 * NC), lambda b: (0, 0)),
            pl.BlockSpec((1, 2 * NC), lambda b: (0, 0)),
        ],
        out_specs=pl.BlockSpec((1, 1, 2 * NC), lambda b: (b, 0, 0)),
        compiler_params=pltpu.CompilerParams(dimension_semantics=("parallel",)),
    )(cols, ws2, _pair2(stem_b), we2, _pair2(exp_b),
      wdw2, _pair2(dw_b), w1_2, _pair2(se_r_b), w2_2, _pair2(se_e_b),
      wp2, _pair2(proj_b), wh2, _pair2(head_b), wf2, _pair2(fc_b))
    return out.reshape(P, 2, NC).reshape(B, NC)
